# Initial kernel scaffold; baseline (speedup 1.0000x reference)
import jax
import jax.numpy as jnp
from jax.experimental import pallas as pl


def kernel(x, W):
    raise NotImplementedError("write your pallas kernel here")


# speedup vs baseline: 4.2596x; 4.2596x over previous
import functools

import jax
import jax.numpy as jnp
from jax import lax
from jax.experimental import pallas as pl
from jax.experimental.pallas import tpu as pltpu
from jax.experimental.pallas import tpu_sc as plsc

N_TOK = 2048
D = 768
NF = 16384
K = 32

BT = 256
BF = 1024
NFB = NF // BF
CHUNK = 128
NCHUNK = NF // CHUNK

NW = 32
TPW = N_TOK // NW
NEG_INF = float("-inf")


def _mm_body(x_ref, w_ref, raw_ref, t0_ref, m_scr):
    j = pl.program_id(1)
    r = lax.dot_general(
        x_ref[...], w_ref[...], (((1,), (1,)), ((), ())),
        preferred_element_type=jnp.float32)
    raw_ref[...] = r
    cm = jnp.max(r.reshape(BT, BF // CHUNK, CHUNK), axis=2)
    m_scr[j] = cm

    @pl.when(j == NFB - 1)
    def _():
        m0 = jnp.concatenate([m_scr[jj] for jj in range(NFB)], axis=-1)

        def it(_, carry):
            m, _ = carry
            g = jnp.max(m, axis=1, keepdims=True)
            return jnp.where(m >= g, NEG_INF, m), g
        _, g = lax.fori_loop(
            0, K, it, (m0, jnp.zeros((BT, 1), jnp.float32)))
        t0_ref[...] = jnp.broadcast_to(g, (BT, 16))


def _matmul_threshold(x, w):
    return pl.pallas_call(
        _mm_body,
        grid=(N_TOK // BT, NFB),
        in_specs=[
            pl.BlockSpec((BT, D), lambda i, j: (i, 0)),
            pl.BlockSpec((BF, D), lambda i, j: (j, 0)),
        ],
        out_specs=[
            pl.BlockSpec((BT, BF), lambda i, j: (i, j)),
            pl.BlockSpec((BT, 16), lambda i, j: (i, 0)),
        ],
        out_shape=[
            jax.ShapeDtypeStruct((N_TOK, NF), jnp.float32),
            jax.ShapeDtypeStruct((N_TOK, 16), jnp.float32),
        ],
        scratch_shapes=[pltpu.VMEM((NFB, BT, BF // CHUNK), jnp.float32)],
        compiler_params=pltpu.CompilerParams(
            dimension_semantics=("parallel", "arbitrary")),
    )(x, w)


def _merge16_desc(ak, av, bk, bv):
    rk = lax.rev(bk, (0,))
    rv = lax.rev(bv, (0,))
    c = ak >= rk
    hk = jnp.where(c, ak, rk)
    hv = jnp.where(c, av, rv)
    lk = jnp.where(c, rk, ak)
    lv = jnp.where(c, rv, av)
    hk, hv = plsc.sort_key_val(hk, hv, descending=True)
    lk, lv = plsc.sort_key_val(lk, lv, descending=True)
    return hk, hv, lk, lv


def _topk_body(raw, t0rep, wv_out, wi_out,
               t0v, rb0, rb1, svv, svi, wvb, wib, sem0, sem1):
    wid = lax.axis_index("s") * 2 + lax.axis_index("c")
    base = wid * TPW
    pltpu.sync_copy(t0rep.at[pl.ds(base, TPW), :], t0v)

    lane = lax.iota(jnp.int32, 16)
    zeros_i = jnp.zeros((16,), jnp.int32)
    neginf_v = jnp.full((16,), NEG_INF, jnp.float32)

    rbufs = (rb0, rb1)
    sems = (sem0, sem1)

    def start_row(tok, b):
        pltpu.async_copy(raw.at[base + tok], rbufs[b], sems[b])

    def wait_row(tok, b):
        pltpu.make_async_copy(raw.at[base + tok], rbufs[b], sems[b]).wait()

    def process(tok, b):
        wait_row(tok, b)

        @pl.when(tok + 1 < TPW)
        def _():
            start_row(tok + 1, (b + 1) % 2)

        rb = rbufs[b]
        thr = t0v[tok]

        def filt(c, carry):
            offv, idxv = carry
            v = rb[pl.ds(c * 16, 16)]
            m = v >= thr
            mi = jnp.where(m, jnp.int32(1), jnp.int32(0))
            csum = plsc.cumsum(mi)
            pos = offv + csum - 1
            plsc.store_scatter(svv, [pos], v, mask=m)
            plsc.store_scatter(svi, [pos], idxv, mask=m)
            pc = plsc.all_reduce_population_count(m)
            return offv + pc, idxv + 16

        offv, _ = lax.fori_loop(0, NF // 16, filt, (zeros_i, lane))
        s = lax.reduce_max(offv, (0,))
        s_al = (s // 16) * 16
        padmask = lane >= (s - s_al)
        plsc.store_scatter(svv, [lane + s_al], neginf_v, mask=padmask)
        nv = s_al // 16 + 1

        def sel(j, carry):
            t0k, t0val, t1k, t1val = carry
            ck = svv[pl.ds(j * 16, 16)]
            cv = svi[pl.ds(j * 16, 16)]
            ck, cv = plsc.sort_key_val(ck, cv, descending=True)
            t0k, t0val, lk, lv = _merge16_desc(t0k, t0val, ck, cv)
            t1k, t1val, _, _ = _merge16_desc(t1k, t1val, lk, lv)
            return t0k, t0val, t1k, t1val

        t0k, t0val, t1k, t1val = lax.fori_loop(
            0, nv, sel, (neginf_v, zeros_i, neginf_v, zeros_i))

        wvb[pl.ds(tok * K, 16)] = t0k
        wvb[pl.ds(tok * K + 16, 16)] = t1k
        wib[pl.ds(tok * K, 16)] = t0val
        wib[pl.ds(tok * K + 16, 16)] = t1val

    start_row(0, 0)

    def pair(i, carry):
        process(2 * i, 0)
        process(2 * i + 1, 1)
        return carry

    lax.fori_loop(0, TPW // 2, pair, 0)

    pltpu.sync_copy(wvb, wv_out.at[pl.ds(base * K, TPW * K)])
    pltpu.sync_copy(wib, wi_out.at[pl.ds(base * K, TPW * K)])


def _topk(raw, t0rep):
    mesh = plsc.VectorSubcoreMesh(core_axis_name="c", subcore_axis_name="s")
    fn = pl.kernel(
        _topk_body,
        out_type=[
            jax.ShapeDtypeStruct((N_TOK * K,), jnp.float32),
            jax.ShapeDtypeStruct((N_TOK * K,), jnp.int32),
        ],
        mesh=mesh,
        compiler_params=pltpu.CompilerParams(needs_layout_passes=False),
        scratch_types=[
            pltpu.VMEM((TPW, 16), jnp.float32),
            pltpu.VMEM((NF,), jnp.float32),
            pltpu.VMEM((NF,), jnp.float32),
            pltpu.VMEM((NF + 16,), jnp.float32),
            pltpu.VMEM((NF + 16,), jnp.int32),
            pltpu.VMEM((TPW * K,), jnp.float32),
            pltpu.VMEM((TPW * K,), jnp.int32),
            pltpu.SemaphoreType.DMA,
            pltpu.SemaphoreType.DMA,
        ],
    )
    wv, wi = fn(raw, t0rep)
    return wv.reshape(N_TOK, K), wi.reshape(N_TOK, K)


def _recon_body(w_hbm, x_hbm, wv, wi, acts_out, recon_out, lossp_out,
                wvb, wib, ar0, ar1, g0, g1, x0, x1, r0, r1, lb,
                semg0, semg1, semx0, semx1, sema0, sema1, semr0, semr1):
    wid = lax.axis_index("s") * 2 + lax.axis_index("c")
    base = wid * TPW
    pltpu.sync_copy(wv.at[pl.ds(base, TPW), :], wvb)
    pltpu.sync_copy(wi.at[pl.ds(base, TPW), :], wib)

    arows = (ar0, ar1)
    gbufs = (g0, g1)
    xbufs = (x0, x1)
    rbufs = (r0, r1)
    semg = (semg0, semg1)
    semx = (semx0, semx1)
    sema = (sema0, sema1)
    semr = (semr0, semr1)

    zv = jnp.zeros((16,), jnp.float32)

    def zloop(i, carry):
        ar0[pl.ds(i * 16, 16)] = zv
        ar1[pl.ds(i * 16, 16)] = zv
        return carry
    lax.fori_loop(0, NF // 16, zloop, 0)

    def idx_chunks(tok):
        return (wib[tok, pl.ds(0, 16)], wib[tok, pl.ds(16, 16)])

    def start_inputs(tok, b):
        i0, i1 = idx_chunks(tok)
        pltpu.async_copy(w_hbm.at[i0], gbufs[b].at[pl.ds(0, 16), :], semg[b])
        pltpu.async_copy(w_hbm.at[i1], gbufs[b].at[pl.ds(16, 16), :], semg[b])
        pltpu.async_copy(x_hbm.at[base + tok], xbufs[b], semx[b])

    def wait_inputs(tok, b):
        i0, i1 = idx_chunks(tok)
        pltpu.make_async_copy(
            w_hbm.at[i0], gbufs[b].at[pl.ds(0, 16), :], semg[b]).wait()
        pltpu.make_async_copy(
            w_hbm.at[i1], gbufs[b].at[pl.ds(16, 16), :], semg[b]).wait()
        pltpu.make_async_copy(x_hbm.at[base + tok], xbufs[b], semx[b]).wait()

    def process(tok, b, lacc):
        ar, gb, xb, rb = arows[b], gbufs[b], xbufs[b], rbufs[b]
        wait_inputs(tok, b)

        @pl.when(tok + 1 < TPW)
        def _():
            start_inputs(tok + 1, (b + 1) % 2)

        @pl.when(tok >= 2)
        def _():
            pltpu.make_async_copy(
                ar, acts_out.at[base + tok - 2], sema[b]).wait()
            pltpu.make_async_copy(
                rb, recon_out.at[base + tok - 2], semr[b]).wait()
            p0, p1 = idx_chunks(tok - 2)
            plsc.store_scatter(ar, [p0], zv)
            plsc.store_scatter(ar, [p1], zv)

        i0, i1 = idx_chunks(tok)
        v0 = wvb[tok, pl.ds(0, 16)]
        v1 = wvb[tok, pl.ds(16, 16)]
        plsc.store_scatter(ar, [i0], v0)
        plsc.store_scatter(ar, [i1], v1)
        pltpu.async_copy(ar, acts_out.at[base + tok], sema[b])

        splats = ([jnp.full((16,), v0[k], jnp.float32) for k in range(16)]
                  + [jnp.full((16,), v1[k], jnp.float32) for k in range(16)])

        def jloop(j, lacc_):
            acc = splats[0] * gb[0, pl.ds(j * 16, 16)]
            for k in range(1, K):
                acc = acc + splats[k] * gb[k, pl.ds(j * 16, 16)]
            rb[pl.ds(j * 16, 16)] = acc
            d = acc - xb[pl.ds(j * 16, 16)]
            return lacc_ + d * d

        lacc = lax.fori_loop(0, D // 16, jloop, lacc)
        pltpu.async_copy(rb, recon_out.at[base + tok], semr[b])
        return lacc

    start_inputs(0, 0)

    def pair(i, lacc):
        lacc = process(2 * i, 0, lacc)
        lacc = process(2 * i + 1, 1, lacc)
        return lacc

    lacc = lax.fori_loop(0, TPW // 2, pair, zv)

    pltpu.make_async_copy(ar0, acts_out.at[base + TPW - 2], sema0).wait()
    pltpu.make_async_copy(r0, recon_out.at[base + TPW - 2], semr0).wait()
    pltpu.make_async_copy(ar1, acts_out.at[base + TPW - 1], sema1).wait()
    pltpu.make_async_copy(r1, recon_out.at[base + TPW - 1], semr1).wait()

    lb[...] = lacc
    pltpu.sync_copy(lb, lossp_out.at[wid])


def _recon(w, x, wv, wi):
    mesh = plsc.VectorSubcoreMesh(core_axis_name="c", subcore_axis_name="s")
    fn = pl.kernel(
        _recon_body,
        out_type=[
            jax.ShapeDtypeStruct((N_TOK, NF), jnp.float32),
            jax.ShapeDtypeStruct((N_TOK, D), jnp.float32),
            jax.ShapeDtypeStruct((NW, 16), jnp.float32),
        ],
        mesh=mesh,
        compiler_params=pltpu.CompilerParams(needs_layout_passes=False),
        scratch_types=[
            pltpu.VMEM((TPW, K), jnp.float32),
            pltpu.VMEM((TPW, K), jnp.int32),
            pltpu.VMEM((NF,), jnp.float32),
            pltpu.VMEM((NF,), jnp.float32),
            pltpu.VMEM((K, D), jnp.float32),
            pltpu.VMEM((K, D), jnp.float32),
            pltpu.VMEM((D,), jnp.float32),
            pltpu.VMEM((D,), jnp.float32),
            pltpu.VMEM((D,), jnp.float32),
            pltpu.VMEM((D,), jnp.float32),
            pltpu.VMEM((16,), jnp.float32),
        ] + [pltpu.SemaphoreType.DMA] * 8,
    )
    return fn(w, x, wv, wi)


def _loss_body(p_ref, o_ref):
    o_ref[...] = (jnp.sum(p_ref[...]) / jnp.float32(N_TOK)).reshape(1, 1)


def _loss_sum(partials):
    return pl.pallas_call(
        _loss_body,
        out_shape=jax.ShapeDtypeStruct((1, 1), jnp.float32),
    )(partials)


def kernel(x, W):
    raw, t0rep = _matmul_threshold(x, W)
    wv, wi = _topk(raw, t0rep)
    acts, recon, lossp = _recon(W, x, wv, wi)
    loss = _loss_sum(lossp).reshape(())
    return (loss, recon, acts)


# speedup vs baseline: 4.6768x; 1.0979x over previous
import functools

import jax
import jax.numpy as jnp
from jax import lax
from jax.experimental import pallas as pl
from jax.experimental.pallas import tpu as pltpu
from jax.experimental.pallas import tpu_sc as plsc

N_TOK = 2048
D = 768
NF = 16384
K = 32

BT = 256
BF = 1024
NFB = NF // BF
CHUNK = 128
NCHUNK = NF // CHUNK

NW = 32
TPW = N_TOK // NW
NEG_INF = float("-inf")


def _mm_body(x_ref, w_ref, raw_ref, t0_ref, m_scr):
    j = pl.program_id(1)
    r = lax.dot_general(
        x_ref[...], w_ref[...], (((1,), (1,)), ((), ())),
        preferred_element_type=jnp.float32)
    raw_ref[...] = r
    cm = jnp.max(r.reshape(BT, BF // CHUNK, CHUNK), axis=2)
    m_scr[j] = cm

    @pl.when(j == NFB - 1)
    def _():
        m0 = jnp.concatenate([m_scr[jj] for jj in range(NFB)], axis=-1)

        def it(_, carry):
            m, _ = carry
            g = jnp.max(m, axis=1, keepdims=True)
            return jnp.where(m >= g, NEG_INF, m), g
        _, g = lax.fori_loop(
            0, K, it, (m0, jnp.zeros((BT, 1), jnp.float32)))
        t0_ref[...] = jnp.broadcast_to(g, (BT, 16))


def _matmul_threshold(x, w):
    return pl.pallas_call(
        _mm_body,
        grid=(N_TOK // BT, NFB),
        in_specs=[
            pl.BlockSpec((BT, D), lambda i, j: (i, 0)),
            pl.BlockSpec((BF, D), lambda i, j: (j, 0)),
        ],
        out_specs=[
            pl.BlockSpec((BT, BF), lambda i, j: (i, j)),
            pl.BlockSpec((BT, 16), lambda i, j: (i, 0)),
        ],
        out_shape=[
            jax.ShapeDtypeStruct((N_TOK, NF), jnp.float32),
            jax.ShapeDtypeStruct((N_TOK, 16), jnp.float32),
        ],
        scratch_shapes=[pltpu.VMEM((NFB, BT, BF // CHUNK), jnp.float32)],
        compiler_params=pltpu.CompilerParams(
            dimension_semantics=("parallel", "arbitrary")),
    )(x, w)


def _merge16_desc(ak, av, bk, bv):
    rk = lax.rev(bk, (0,))
    rv = lax.rev(bv, (0,))
    c = ak >= rk
    hk = jnp.where(c, ak, rk)
    hv = jnp.where(c, av, rv)
    lk = jnp.where(c, rk, ak)
    lv = jnp.where(c, rv, av)
    hk, hv = plsc.sort_key_val(hk, hv, descending=True)
    lk, lv = plsc.sort_key_val(lk, lv, descending=True)
    return hk, hv, lk, lv


def _topk_body(raw, t0rep, wv_out, wi_out,
               t0v, rb0, rb1, svi, wvb, wib, sem0, sem1):
    wid = lax.axis_index("s") * 2 + lax.axis_index("c")
    base = wid * TPW
    pltpu.sync_copy(t0rep.at[pl.ds(base, TPW), :], t0v)

    lane = lax.iota(jnp.int32, 16)
    zeros_i = jnp.zeros((16,), jnp.int32)
    neginf_v = jnp.full((16,), NEG_INF, jnp.float32)

    rbufs = (rb0, rb1)
    sems = (sem0, sem1)
    rb0[pl.ds(NF, 16)] = neginf_v
    rb1[pl.ds(NF, 16)] = neginf_v

    def start_row(tok, b):
        pltpu.async_copy(raw.at[base + tok], rbufs[b].at[pl.ds(0, NF)],
                         sems[b])

    def wait_row(tok, b):
        pltpu.make_async_copy(raw.at[base + tok], rbufs[b].at[pl.ds(0, NF)],
                              sems[b]).wait()

    def process(tok, b):
        wait_row(tok, b)

        @pl.when(tok + 1 < TPW)
        def _():
            start_row(tok + 1, (b + 1) % 2)

        rb = rbufs[b]
        thr = t0v[tok]

        def filt(c, off_s):
            v = rb[pl.ds(c * 16, 16)]
            m = v >= thr
            idxv = lane + c * 16
            plsc.store_compressed(svi.at[pl.ds(off_s, 16)], idxv, mask=m)
            pc = plsc.all_reduce_population_count(m)
            return off_s + pc[0]

        s = lax.fori_loop(0, NF // 16, filt, jnp.int32(0))
        svi[pl.ds(s, 16)] = lane + NF
        nv = s // 16 + 1

        def sel(j, carry):
            t0k, t0val, t1k, t1val = carry
            cv = svi[pl.ds(j * 16, 16)]
            ck = plsc.load_gather(rb, [cv])
            ck, cv = plsc.sort_key_val(ck, cv, descending=True)
            t0k, t0val, lk, lv = _merge16_desc(t0k, t0val, ck, cv)
            t1k, t1val, _, _ = _merge16_desc(t1k, t1val, lk, lv)
            return t0k, t0val, t1k, t1val

        t0k, t0val, t1k, t1val = lax.fori_loop(
            0, nv, sel, (neginf_v, zeros_i, neginf_v, zeros_i))

        wvb[pl.ds(tok * K, 16)] = t0k
        wvb[pl.ds(tok * K + 16, 16)] = t1k
        wib[pl.ds(tok * K, 16)] = t0val
        wib[pl.ds(tok * K + 16, 16)] = t1val

    start_row(0, 0)

    def pair(i, carry):
        process(2 * i, 0)
        process(2 * i + 1, 1)
        return carry

    lax.fori_loop(0, TPW // 2, pair, 0)

    pltpu.sync_copy(wvb, wv_out.at[pl.ds(base * K, TPW * K)])
    pltpu.sync_copy(wib, wi_out.at[pl.ds(base * K, TPW * K)])


def _topk(raw, t0rep):
    mesh = plsc.VectorSubcoreMesh(core_axis_name="c", subcore_axis_name="s")
    fn = pl.kernel(
        _topk_body,
        out_type=[
            jax.ShapeDtypeStruct((N_TOK * K,), jnp.float32),
            jax.ShapeDtypeStruct((N_TOK * K,), jnp.int32),
        ],
        mesh=mesh,
        compiler_params=pltpu.CompilerParams(needs_layout_passes=False),
        scratch_types=[
            pltpu.VMEM((TPW, 16), jnp.float32),
            pltpu.VMEM((NF + 16,), jnp.float32),
            pltpu.VMEM((NF + 16,), jnp.float32),
            pltpu.VMEM((NF + 32,), jnp.int32),
            pltpu.VMEM((TPW * K,), jnp.float32),
            pltpu.VMEM((TPW * K,), jnp.int32),
            pltpu.SemaphoreType.DMA,
            pltpu.SemaphoreType.DMA,
        ],
    )
    wv, wi = fn(raw, t0rep)
    return wv.reshape(N_TOK, K), wi.reshape(N_TOK, K)


def _recon_body(w_hbm, x_hbm, wv, wi, acts_out, recon_out, lossp_out,
                wvb, wib, ar0, ar1, g0, g1, x0, x1, r0, r1, lb,
                semg0, semg1, semx0, semx1, sema0, sema1, semr0, semr1):
    wid = lax.axis_index("s") * 2 + lax.axis_index("c")
    base = wid * TPW
    pltpu.sync_copy(wv.at[pl.ds(base, TPW), :], wvb)
    pltpu.sync_copy(wi.at[pl.ds(base, TPW), :], wib)

    arows = (ar0, ar1)
    gbufs = (g0, g1)
    xbufs = (x0, x1)
    rbufs = (r0, r1)
    semg = (semg0, semg1)
    semx = (semx0, semx1)
    sema = (sema0, sema1)
    semr = (semr0, semr1)

    zv = jnp.zeros((16,), jnp.float32)

    def zloop(i, carry):
        ar0[pl.ds(i * 16, 16)] = zv
        ar1[pl.ds(i * 16, 16)] = zv
        return carry
    lax.fori_loop(0, NF // 16, zloop, 0)

    def idx_chunks(tok):
        return (wib[tok, pl.ds(0, 16)], wib[tok, pl.ds(16, 16)])

    def start_inputs(tok, b):
        i0, i1 = idx_chunks(tok)
        pltpu.async_copy(w_hbm.at[i0], gbufs[b].at[pl.ds(0, 16), :], semg[b])
        pltpu.async_copy(w_hbm.at[i1], gbufs[b].at[pl.ds(16, 16), :], semg[b])
        pltpu.async_copy(x_hbm.at[base + tok], xbufs[b], semx[b])

    def wait_inputs(tok, b):
        i0, i1 = idx_chunks(tok)
        pltpu.make_async_copy(
            w_hbm.at[i0], gbufs[b].at[pl.ds(0, 16), :], semg[b]).wait()
        pltpu.make_async_copy(
            w_hbm.at[i1], gbufs[b].at[pl.ds(16, 16), :], semg[b]).wait()
        pltpu.make_async_copy(x_hbm.at[base + tok], xbufs[b], semx[b]).wait()

    def process(tok, b, lacc):
        ar, gb, xb, rb = arows[b], gbufs[b], xbufs[b], rbufs[b]
        wait_inputs(tok, b)

        @pl.when(tok + 1 < TPW)
        def _():
            start_inputs(tok + 1, (b + 1) % 2)

        @pl.when(tok >= 2)
        def _():
            pltpu.make_async_copy(
                ar, acts_out.at[base + tok - 2], sema[b]).wait()
            pltpu.make_async_copy(
                rb, recon_out.at[base + tok - 2], semr[b]).wait()
            p0, p1 = idx_chunks(tok - 2)
            plsc.store_scatter(ar, [p0], zv)
            plsc.store_scatter(ar, [p1], zv)

        i0, i1 = idx_chunks(tok)
        v0 = wvb[tok, pl.ds(0, 16)]
        v1 = wvb[tok, pl.ds(16, 16)]
        plsc.store_scatter(ar, [i0], v0)
        plsc.store_scatter(ar, [i1], v1)
        pltpu.async_copy(ar, acts_out.at[base + tok], sema[b])

        splats = ([jnp.full((16,), v0[k], jnp.float32) for k in range(16)]
                  + [jnp.full((16,), v1[k], jnp.float32) for k in range(16)])

        def jloop(j, lacc_):
            acc = splats[0] * gb[0, pl.ds(j * 16, 16)]
            for k in range(1, K):
                acc = acc + splats[k] * gb[k, pl.ds(j * 16, 16)]
            rb[pl.ds(j * 16, 16)] = acc
            d = acc - xb[pl.ds(j * 16, 16)]
            return lacc_ + d * d

        lacc = lax.fori_loop(0, D // 16, jloop, lacc)
        pltpu.async_copy(rb, recon_out.at[base + tok], semr[b])
        return lacc

    start_inputs(0, 0)

    def pair(i, lacc):
        lacc = process(2 * i, 0, lacc)
        lacc = process(2 * i + 1, 1, lacc)
        return lacc

    lacc = lax.fori_loop(0, TPW // 2, pair, zv)

    pltpu.make_async_copy(ar0, acts_out.at[base + TPW - 2], sema0).wait()
    pltpu.make_async_copy(r0, recon_out.at[base + TPW - 2], semr0).wait()
    pltpu.make_async_copy(ar1, acts_out.at[base + TPW - 1], sema1).wait()
    pltpu.make_async_copy(r1, recon_out.at[base + TPW - 1], semr1).wait()

    lb[...] = lacc
    pltpu.sync_copy(lb, lossp_out.at[wid])


def _recon(w, x, wv, wi):
    mesh = plsc.VectorSubcoreMesh(core_axis_name="c", subcore_axis_name="s")
    fn = pl.kernel(
        _recon_body,
        out_type=[
            jax.ShapeDtypeStruct((N_TOK, NF), jnp.float32),
            jax.ShapeDtypeStruct((N_TOK, D), jnp.float32),
            jax.ShapeDtypeStruct((NW, 16), jnp.float32),
        ],
        mesh=mesh,
        compiler_params=pltpu.CompilerParams(needs_layout_passes=False),
        scratch_types=[
            pltpu.VMEM((TPW, K), jnp.float32),
            pltpu.VMEM((TPW, K), jnp.int32),
            pltpu.VMEM((NF,), jnp.float32),
            pltpu.VMEM((NF,), jnp.float32),
            pltpu.VMEM((K, D), jnp.float32),
            pltpu.VMEM((K, D), jnp.float32),
            pltpu.VMEM((D,), jnp.float32),
            pltpu.VMEM((D,), jnp.float32),
            pltpu.VMEM((D,), jnp.float32),
            pltpu.VMEM((D,), jnp.float32),
            pltpu.VMEM((16,), jnp.float32),
        ] + [pltpu.SemaphoreType.DMA] * 8,
    )
    return fn(w, x, wv, wi)


def _loss_body(p_ref, o_ref):
    o_ref[...] = (jnp.sum(p_ref[...]) / jnp.float32(N_TOK)).reshape(1, 1)


def _loss_sum(partials):
    return pl.pallas_call(
        _loss_body,
        out_shape=jax.ShapeDtypeStruct((1, 1), jnp.float32),
    )(partials)


def kernel(x, W):
    raw, t0rep = _matmul_threshold(x, W)
    wv, wi = _topk(raw, t0rep)
    acts, recon, lossp = _recon(W, x, wv, wi)
    loss = _loss_sum(lossp).reshape(())
    return (loss, recon, acts)
